# baseline (speedup 1.0000x reference)
import functools

import jax
import jax.numpy as jnp
from jax.experimental import pallas as pl
from jax.experimental.pallas import tpu as pltpu

B, D_IN, H, D_OUT = 2048, 1024, 384, 1024
N_STEPS = H - 1
NB = 1
B_BLK = B // NB
T = H // 3
G = 32
U = 32
U2 = 8
TAIL = (N_STEPS - 2 * T) % U2 or U2


def _dg(a, b, ca, cb):
    return jax.lax.dot_general(a, b, (((ca,), (cb,)), ((), ())),
                               preferred_element_type=jnp.float32)


def _sqr(x):
    r = jnp.maximum(x, 0.0)
    return r * r


def _casc0_body(x_ref, wi_ref, bi_ref, w_ref, b_ref, out_ref, h_ref):
    t = pl.program_id(1)

    @pl.when(t == 0)
    def _init():
        h_ref[...] = _sqr(_dg(wi_ref[...], x_ref[...], 1, 1) + bi_ref[...])

    for u in range(U):
        pre = _dg(w_ref[u, 0], h_ref[:T], 1, 0)
        h_ref[:T] = h_ref[:T] + _sqr(pre + b_ref[u, :T])

    @pl.when(t == pl.num_programs(1) - 1)
    def _final():
        out_ref[...] = h_ref[...]


def _off2_body(h_in_ref, w1_ref, w2_ref, b_ref, out_ref, acc_ref):
    g = pl.program_id(1)

    @pl.when(g == 0)
    def _init():
        acc_ref[...] = h_in_ref[T:]

    for u in range(G):
        pre1 = _dg(w1_ref[u, 0], h_in_ref[:T], 1, 0)
        pre2 = _dg(w2_ref[u, 0], h_in_ref[:T], 1, 0)
        acc_ref[:T] = acc_ref[:T] + _sqr(pre1 + b_ref[u, T:2 * T])
        acc_ref[T:] = acc_ref[T:] + _sqr(pre2 + b_ref[u, 2 * T:])

    @pl.when(g == pl.num_programs(1) - 1)
    def _final():
        out_ref[:T] = h_in_ref[:T]
        out_ref[T:] = acc_ref[...]


def _off1_body(h_in_ref, w_ref, b_ref, out_ref, acc_ref):
    g = pl.program_id(1)

    @pl.when(g == 0)
    def _init():
        acc_ref[...] = h_in_ref[2 * T:]

    for u in range(G):
        pre = _dg(w_ref[u, 0], h_in_ref[:2 * T], 1, 0)
        acc_ref[...] = acc_ref[...] + _sqr(pre + b_ref[u, 2 * T:])

    @pl.when(g == pl.num_programs(1) - 1)
    def _final():
        out_ref[:2 * T] = h_in_ref[:2 * T]
        out_ref[2 * T:] = acc_ref[...]


def _casc1_body(h_in_ref, w_ref, b_ref, out_ref, h_ref):
    t = pl.program_id(1)

    @pl.when(t == 0)
    def _init():
        h_ref[...] = h_in_ref[...]

    for u in range(U):
        pre = _dg(w_ref[u, 0], h_ref[:2 * T], 1, 0)
        h_ref[T:2 * T] = h_ref[T:2 * T] + _sqr(pre + b_ref[u, T:2 * T])

    @pl.when(t == pl.num_programs(1) - 1)
    def _final():
        out_ref[...] = h_ref[...]


def _casc2_body(h_in_ref, w_ref, b_ref, out_ref, h_ref):
    t = pl.program_id(1)

    @pl.when(t == 0)
    def _init():
        h_ref[...] = h_in_ref[...]

    for u in range(U2):
        pre = _dg(w_ref[u, 0], h_ref[...], 1, 0)
        h_ref[2 * T:] = h_ref[2 * T:] + _sqr(pre + b_ref[u, 2 * T:])

    @pl.when(t == pl.num_programs(1) - 1)
    def _final():
        out_ref[...] = h_ref[...]


def _tail_body(h_in_ref, w_ref, b_ref, wo_ref, bo_ref, out_ref, h_ref):
    s = pl.program_id(1)

    @pl.when(s == 0)
    def _init():
        h_ref[...] = h_in_ref[...]

    pre = _dg(w_ref[0, 0], h_ref[...], 1, 0)
    h_ref[2 * T:] = h_ref[2 * T:] + _sqr(pre + b_ref[0, 2 * T:])

    @pl.when(s == pl.num_programs(1) - 1)
    def _final():
        out_ref[...] = _dg(h_ref[...], wo_ref[...], 0, 1) + bo_ref[...]


def _cparams():
    return pltpu.CompilerParams(
        dimension_semantics=("parallel", "arbitrary"))


@functools.partial(jax.jit, static_argnames=("interpret",))
def kernel(x, Wi, bi, W_pad, b_pad, Wo, bo, interpret=False):
    biT = bi.reshape(H, 1)
    bT3 = b_pad.reshape(N_STEPS, H, 1)
    boR = bo.reshape(1, D_OUT)
    W4 = W_pad.reshape(N_STEPS, 3, T, H)
    hspec = pl.BlockSpec((H, B_BLK), lambda c, s: (0, c))
    hshape = jax.ShapeDtypeStruct((H, B), jnp.float32)
    n2 = (N_STEPS - 2 * T - TAIL) // U2

    hT = pl.pallas_call(
        _casc0_body,
        out_shape=hshape,
        grid=(NB, T // U),
        in_specs=[
            pl.BlockSpec((B_BLK, D_IN), lambda c, t: (c, 0)),
            pl.BlockSpec((H, D_IN), lambda c, t: (0, 0)),
            pl.BlockSpec((H, 1), lambda c, t: (0, 0)),
            pl.BlockSpec((U, 1, T, T), lambda c, t: (t, 0, 0, 0)),
            pl.BlockSpec((U, H, 1), lambda c, t: (t, 0, 0)),
        ],
        out_specs=hspec,
        scratch_shapes=[pltpu.VMEM((H, B_BLK), jnp.float32)],
        compiler_params=_cparams(),
        name="fn_casc0",
        interpret=interpret,
    )(x, Wi, biT, W4, bT3)

    hT = pl.pallas_call(
        _off2_body,
        out_shape=hshape,
        grid=(NB, T // G),
        in_specs=[
            hspec,
            pl.BlockSpec((G, 1, T, T), lambda c, g: (g, 1, 0, 0)),
            pl.BlockSpec((G, 1, T, T), lambda c, g: (g, 2, 0, 0)),
            pl.BlockSpec((G, H, 1), lambda c, g: (g, 0, 0)),
        ],
        out_specs=hspec,
        scratch_shapes=[pltpu.VMEM((H - T, B_BLK), jnp.float32)],
        compiler_params=_cparams(),
        name="fn_off0",
        interpret=interpret,
    )(hT, W4, W4, bT3)

    hT = pl.pallas_call(
        _casc1_body,
        out_shape=hshape,
        grid=(NB, T // U),
        in_specs=[
            hspec,
            pl.BlockSpec((U, 1, T, 2 * T), lambda c, t: (T // U + t, 1, 0, 0)),
            pl.BlockSpec((U, H, 1), lambda c, t: (T // U + t, 0, 0)),
        ],
        out_specs=hspec,
        scratch_shapes=[pltpu.VMEM((H, B_BLK), jnp.float32)],
        compiler_params=_cparams(),
        name="fn_casc1",
        interpret=interpret,
    )(hT, W4, bT3)

    hT = pl.pallas_call(
        _off1_body,
        out_shape=hshape,
        grid=(NB, T // G),
        in_specs=[
            hspec,
            pl.BlockSpec((G, 1, T, 2 * T), lambda c, g: (T // G + g, 2, 0, 0)),
            pl.BlockSpec((G, H, 1), lambda c, g: (T // G + g, 0, 0)),
        ],
        out_specs=hspec,
        scratch_shapes=[pltpu.VMEM((H - 2 * T, B_BLK), jnp.float32)],
        compiler_params=_cparams(),
        name="fn_off1",
        interpret=interpret,
    )(hT, W4, bT3)

    hT = pl.pallas_call(
        _casc2_body,
        out_shape=hshape,
        grid=(NB, n2),
        in_specs=[
            hspec,
            pl.BlockSpec((U2, 1, T, H), lambda c, t: (2 * T // U2 + t, 2, 0, 0)),
            pl.BlockSpec((U2, H, 1), lambda c, t: (2 * T // U2 + t, 0, 0)),
        ],
        out_specs=hspec,
        scratch_shapes=[pltpu.VMEM((H, B_BLK), jnp.float32)],
        compiler_params=_cparams(),
        name="fn_casc2",
        interpret=interpret,
    )(hT, W4, bT3)

    t0 = 2 * T + n2 * U2
    return pl.pallas_call(
        _tail_body,
        out_shape=jax.ShapeDtypeStruct((B, D_OUT), jnp.float32),
        grid=(NB, TAIL),
        in_specs=[
            hspec,
            pl.BlockSpec((1, 1, T, H), lambda c, s: (s + t0, 2, 0, 0)),
            pl.BlockSpec((1, H, 1), lambda c, s: (s + t0, 0, 0)),
            pl.BlockSpec((D_OUT, H), lambda c, s: (0, 0)),
            pl.BlockSpec((1, D_OUT), lambda c, s: (0, 0)),
        ],
        out_specs=pl.BlockSpec((B_BLK, D_OUT), lambda c, s: (c, 0)),
        scratch_shapes=[pltpu.VMEM((H, B_BLK), jnp.float32)],
        compiler_params=_cparams(),
        name="fn_tail_out",
        interpret=interpret,
    )(hT, W4, bT3, Wo, boR)


# speedup vs baseline: 1.0215x; 1.0215x over previous
import functools

import jax
import jax.numpy as jnp
from jax.experimental import pallas as pl
from jax.experimental.pallas import tpu as pltpu

B, D_IN, H, D_OUT = 2048, 1024, 384, 1024
N_STEPS = H - 1
NB = 1
B_BLK = B // NB
T = H // 3
G = 16
U = 16
U2 = 8
TAIL = (N_STEPS - 2 * T) % U2 or U2


def _dg(a, b, ca, cb):
    return jax.lax.dot_general(a, b, (((ca,), (cb,)), ((), ())),
                               preferred_element_type=jnp.float32)


def _sqr(x):
    r = jnp.maximum(x, 0.0)
    return r * r


def _casc0_body(x_ref, wi_ref, bi_ref, w_ref, b_ref, out_ref, h_ref):
    t = pl.program_id(1)

    @pl.when(t == 0)
    def _init():
        h_ref[...] = _sqr(_dg(wi_ref[...], x_ref[...], 1, 1) + bi_ref[...])

    for u in range(U):
        pre = _dg(w_ref[u, 0], h_ref[:T], 1, 0)
        h_ref[:T] = h_ref[:T] + _sqr(pre + b_ref[u, :T])

    @pl.when(t == pl.num_programs(1) - 1)
    def _final():
        out_ref[...] = h_ref[...]


def _off2_body(h_in_ref, w1_ref, w2_ref, b_ref, out_ref, acc_ref):
    g = pl.program_id(1)

    @pl.when(g == 0)
    def _init():
        acc_ref[...] = h_in_ref[T:]

    for u in range(G):
        pre1 = _dg(w1_ref[u, 0], h_in_ref[:T], 1, 0)
        pre2 = _dg(w2_ref[u, 0], h_in_ref[:T], 1, 0)
        acc_ref[:T] = acc_ref[:T] + _sqr(pre1 + b_ref[u, T:2 * T])
        acc_ref[T:] = acc_ref[T:] + _sqr(pre2 + b_ref[u, 2 * T:])

    @pl.when(g == pl.num_programs(1) - 1)
    def _final():
        out_ref[:T] = h_in_ref[:T]
        out_ref[T:] = acc_ref[...]


def _off1_body(h_in_ref, w_ref, b_ref, out_ref, acc_ref):
    g = pl.program_id(1)

    @pl.when(g == 0)
    def _init():
        acc_ref[...] = h_in_ref[2 * T:]

    for u in range(G):
        pre = _dg(w_ref[u, 0], h_in_ref[:2 * T], 1, 0)
        acc_ref[...] = acc_ref[...] + _sqr(pre + b_ref[u, 2 * T:])

    @pl.when(g == pl.num_programs(1) - 1)
    def _final():
        out_ref[:2 * T] = h_in_ref[:2 * T]
        out_ref[2 * T:] = acc_ref[...]


def _casc1_body(h_in_ref, w_ref, b_ref, out_ref, h_ref):
    t = pl.program_id(1)

    @pl.when(t == 0)
    def _init():
        h_ref[...] = h_in_ref[...]

    for u in range(U):
        pre = _dg(w_ref[u, 0], h_ref[:2 * T], 1, 0)
        h_ref[T:2 * T] = h_ref[T:2 * T] + _sqr(pre + b_ref[u, T:2 * T])

    @pl.when(t == pl.num_programs(1) - 1)
    def _final():
        out_ref[...] = h_ref[...]


def _casc2_body(h_in_ref, w_ref, b_ref, out_ref, h_ref):
    t = pl.program_id(1)

    @pl.when(t == 0)
    def _init():
        h_ref[...] = h_in_ref[...]

    for u in range(U2):
        pre = _dg(w_ref[u, 0], h_ref[...], 1, 0)
        h_ref[2 * T:] = h_ref[2 * T:] + _sqr(pre + b_ref[u, 2 * T:])

    @pl.when(t == pl.num_programs(1) - 1)
    def _final():
        out_ref[...] = h_ref[...]


def _tail_body(h_in_ref, w_ref, b_ref, wo_ref, bo_ref, out_ref, h_ref):
    s = pl.program_id(1)

    @pl.when(s == 0)
    def _init():
        h_ref[...] = h_in_ref[...]

    pre = _dg(w_ref[0, 0], h_ref[...], 1, 0)
    h_ref[2 * T:] = h_ref[2 * T:] + _sqr(pre + b_ref[0, 2 * T:])

    @pl.when(s == pl.num_programs(1) - 1)
    def _final():
        out_ref[...] = _dg(h_ref[...], wo_ref[...], 0, 1) + bo_ref[...]


def _cparams():
    return pltpu.CompilerParams(
        dimension_semantics=("parallel", "arbitrary"))


@functools.partial(jax.jit, static_argnames=("interpret",))
def kernel(x, Wi, bi, W_pad, b_pad, Wo, bo, interpret=False):
    biT = bi.reshape(H, 1)
    bT3 = b_pad.reshape(N_STEPS, H, 1)
    boR = bo.reshape(1, D_OUT)
    W4 = W_pad.reshape(N_STEPS, 3, T, H)
    hspec = pl.BlockSpec((H, B_BLK), lambda c, s: (0, c))
    hshape = jax.ShapeDtypeStruct((H, B), jnp.float32)
    n2 = (N_STEPS - 2 * T - TAIL) // U2

    hT = pl.pallas_call(
        _casc0_body,
        out_shape=hshape,
        grid=(NB, T // U),
        in_specs=[
            pl.BlockSpec((B_BLK, D_IN), lambda c, t: (c, 0)),
            pl.BlockSpec((H, D_IN), lambda c, t: (0, 0)),
            pl.BlockSpec((H, 1), lambda c, t: (0, 0)),
            pl.BlockSpec((U, 1, T, T), lambda c, t: (t, 0, 0, 0)),
            pl.BlockSpec((U, H, 1), lambda c, t: (t, 0, 0)),
        ],
        out_specs=hspec,
        scratch_shapes=[pltpu.VMEM((H, B_BLK), jnp.float32)],
        compiler_params=_cparams(),
        name="fn_casc0",
        interpret=interpret,
    )(x, Wi, biT, W4, bT3)

    hT = pl.pallas_call(
        _off2_body,
        out_shape=hshape,
        grid=(NB, T // G),
        in_specs=[
            hspec,
            pl.BlockSpec((G, 1, T, T), lambda c, g: (g, 1, 0, 0)),
            pl.BlockSpec((G, 1, T, T), lambda c, g: (g, 2, 0, 0)),
            pl.BlockSpec((G, H, 1), lambda c, g: (g, 0, 0)),
        ],
        out_specs=hspec,
        scratch_shapes=[pltpu.VMEM((H - T, B_BLK), jnp.float32)],
        compiler_params=_cparams(),
        name="fn_off0",
        interpret=interpret,
    )(hT, W4, W4, bT3)

    hT = pl.pallas_call(
        _casc1_body,
        out_shape=hshape,
        grid=(NB, T // U),
        in_specs=[
            hspec,
            pl.BlockSpec((U, 1, T, 2 * T), lambda c, t: (T // U + t, 1, 0, 0)),
            pl.BlockSpec((U, H, 1), lambda c, t: (T // U + t, 0, 0)),
        ],
        out_specs=hspec,
        scratch_shapes=[pltpu.VMEM((H, B_BLK), jnp.float32)],
        compiler_params=_cparams(),
        name="fn_casc1",
        interpret=interpret,
    )(hT, W4, bT3)

    hT = pl.pallas_call(
        _off1_body,
        out_shape=hshape,
        grid=(NB, T // G),
        in_specs=[
            hspec,
            pl.BlockSpec((G, 1, T, 2 * T), lambda c, g: (T // G + g, 2, 0, 0)),
            pl.BlockSpec((G, H, 1), lambda c, g: (T // G + g, 0, 0)),
        ],
        out_specs=hspec,
        scratch_shapes=[pltpu.VMEM((H - 2 * T, B_BLK), jnp.float32)],
        compiler_params=_cparams(),
        name="fn_off1",
        interpret=interpret,
    )(hT, W4, bT3)

    hT = pl.pallas_call(
        _casc2_body,
        out_shape=hshape,
        grid=(NB, n2),
        in_specs=[
            hspec,
            pl.BlockSpec((U2, 1, T, H), lambda c, t: (2 * T // U2 + t, 2, 0, 0)),
            pl.BlockSpec((U2, H, 1), lambda c, t: (2 * T // U2 + t, 0, 0)),
        ],
        out_specs=hspec,
        scratch_shapes=[pltpu.VMEM((H, B_BLK), jnp.float32)],
        compiler_params=_cparams(),
        name="fn_casc2",
        interpret=interpret,
    )(hT, W4, bT3)

    t0 = 2 * T + n2 * U2
    return pl.pallas_call(
        _tail_body,
        out_shape=jax.ShapeDtypeStruct((B, D_OUT), jnp.float32),
        grid=(NB, TAIL),
        in_specs=[
            hspec,
            pl.BlockSpec((1, 1, T, H), lambda c, s: (s + t0, 2, 0, 0)),
            pl.BlockSpec((1, H, 1), lambda c, s: (s + t0, 0, 0)),
            pl.BlockSpec((D_OUT, H), lambda c, s: (0, 0)),
            pl.BlockSpec((1, D_OUT), lambda c, s: (0, 0)),
        ],
        out_specs=pl.BlockSpec((B_BLK, D_OUT), lambda c, s: (c, 0)),
        scratch_shapes=[pltpu.VMEM((H, B_BLK), jnp.float32)],
        compiler_params=_cparams(),
        name="fn_tail_out",
        interpret=interpret,
    )(hT, W4, bT3, Wo, boR)
